# baseline (speedup 1.0000x reference)
import functools

import jax
import jax.numpy as jnp
from jax import lax
from jax.experimental import pallas as pl
from jax.experimental.pallas import tpu as pltpu

_MASK_THR = 0.45
_SIGMA = 2.0


def _sweep_body(T, nb, n, ma_ref, mb_ref, sa_ref, sb_ref, la_ref, lb_ref,
                m_ref, c_scr):
    b = pl.program_id(0)
    a = pl.program_id(1)

    @pl.when(a == 0)
    def _():
        m_ref[...] = jnp.zeros_like(m_ref)
        c_scr[:, pl.ds(b * T, T)] = jnp.zeros((1, T), jnp.float32)

    def tile(tri, edge):
        inter = lax.dot_general(ma_ref[...], mb_ref[...],
                                (((1,), (1,)), ((), ())),
                                preferred_element_type=jnp.float32)
        sa = sa_ref[0, 0, :]
        sb = sb_ref[0, 0, :]
        iou = inter / (sa[:, None] + sb[None, :] - inter)
        valid = la_ref[0, 0, :][:, None] == lb_ref[0, 0, :][None, :]
        if tri or edge:
            rj = lax.broadcasted_iota(jnp.int32, inter.shape, 1)
        if tri:
            ri = lax.broadcasted_iota(jnp.int32, inter.shape, 0)
            valid &= ri < rj
        if edge:
            valid &= (b * T + rj) < n
        d = jnp.where(valid, iou, 0.0)
        csl = c_scr[:, pl.ds(b * T, T)]
        csl = jnp.maximum(csl, jnp.max(d, axis=0)[None, :])
        c_scr[:, pl.ds(b * T, T)] = csl
        ca = c_scr[0, pl.ds(a * T, T)]
        term = d * d - (ca * ca)[:, None]
        m_ref[0, 0, :] = jnp.maximum(m_ref[0, 0, :], jnp.max(term, axis=0))

    edge_col = nb * T > n

    @pl.when(a < b)
    def _():
        if edge_col:
            @pl.when(b < nb - 1)
            def _():
                tile(False, False)

            @pl.when(b == nb - 1)
            def _():
                tile(False, True)
        else:
            tile(False, False)

    @pl.when(a == b)
    def _():
        if edge_col:
            @pl.when(b < nb - 1)
            def _():
                tile(True, False)

            @pl.when(b == nb - 1)
            def _():
                tile(True, True)
        else:
            tile(True, False)

    @pl.when(a == nb - 1)
    def _():
        m_ref[...] = jnp.exp(-_SIGMA * m_ref[...])


def _to_bf16_body(x_ref, o_ref):
    o_ref[...] = x_ref[...].astype(jnp.bfloat16)


def _to_bf16(x, T):
    n, hw = x.shape
    nb = (n + T - 1) // T
    return pl.pallas_call(
        _to_bf16_body,
        grid=(nb,),
        in_specs=[pl.BlockSpec((T, hw), lambda i: (i, 0))],
        out_specs=pl.BlockSpec((T, hw), lambda i: (i, 0)),
        out_shape=jax.ShapeDtypeStruct((n, hw), jnp.bfloat16),
    )(x)


def _nms_core(masks, sums, labels, T):
    n, hw = masks.shape
    nb = (n + T - 1) // T
    P = nb * T
    grid = (nb, nb)
    sums3 = sums.reshape(nb, 1, T)
    labels3 = labels.reshape(nb, 1, T)
    mspec_a = pl.BlockSpec((T, hw), lambda b, a: (jnp.minimum(a, b), 0))
    mspec_b = pl.BlockSpec((T, hw), lambda b, a: (b, 0))
    vspec_a = pl.BlockSpec((1, 1, T), lambda b, a: (jnp.minimum(a, b), 0, 0))
    vspec_b = pl.BlockSpec((1, 1, T), lambda b, a: (b, 0, 0))
    params = pltpu.CompilerParams(dimension_semantics=("arbitrary", "arbitrary"))
    coeff = pl.pallas_call(
        functools.partial(_sweep_body, T, nb, n),
        grid=grid,
        in_specs=[mspec_a, mspec_b, vspec_a, vspec_b, vspec_a, vspec_b],
        out_specs=pl.BlockSpec((1, 1, T), lambda b, a: (b, 0, 0)),
        out_shape=jax.ShapeDtypeStruct((nb, 1, T), jnp.float32),
        scratch_shapes=[pltpu.VMEM((1, P), jnp.float32)],
        compiler_params=params,
    )(masks, masks, sums3, sums3, labels3, labels3)
    return coeff.reshape(P)


def kernel(seg_preds, cate_scores, cate_labels):
    n, h, w = seg_preds.shape
    hw = h * w
    seg_masks_b = seg_preds > _MASK_THR
    seg_masks_f = seg_masks_b.astype(jnp.float32)
    sum_masks = seg_masks_f.reshape(n, -1).sum(axis=1)
    seg_scores = (seg_preds * seg_masks_f).reshape(n, -1).sum(axis=1) / sum_masks
    cs = cate_scores * seg_scores
    sort_inds = jnp.argsort(-cs)

    T = 640
    P = ((n + T - 1) // T) * T
    pad = P - n

    masks_s = jnp.take(seg_masks_f, sort_inds, axis=0).reshape(n, hw)
    masks_s = _to_bf16(masks_s, T)
    seg_preds_s = jnp.take(seg_preds, sort_inds, axis=0)
    sums_p = jnp.pad(jnp.take(sum_masks, sort_inds), (0, pad),
                     constant_values=1.0)
    labels_p = jnp.pad(jnp.take(cate_labels, sort_inds), (0, pad),
                       constant_values=-1)
    coeff = _nms_core(masks_s, sums_p, labels_p, T)
    scores_s = jnp.take(cs, sort_inds)
    return (seg_preds_s,
            scores_s * coeff[:n],
            jnp.take(cate_labels, sort_inds))


# speedup vs baseline: 1.5064x; 1.5064x over previous
import functools

import jax
import jax.numpy as jnp
from jax import lax
from jax.experimental import pallas as pl
from jax.experimental.pallas import tpu as pltpu

_MASK_THR = 0.45
_SIGMA = 2.0


def _sweep_body(T, nb, n, ma_ref, mb_ref, sa_ref, sb_ref, la_ref, lb_ref,
                m_ref, c_scr):
    b = pl.program_id(0)
    a = pl.program_id(1)

    @pl.when(a == 0)
    def _():
        m_ref[...] = jnp.zeros_like(m_ref)
        c_scr[:, pl.ds(b * T, T)] = jnp.zeros((1, T), jnp.float32)

    def tile(tri, edge):
        inter = lax.dot_general(ma_ref[...], mb_ref[...],
                                (((1,), (1,)), ((), ())),
                                preferred_element_type=jnp.float32)
        sa = sa_ref[0, 0, :]
        sb = sb_ref[0, 0, :]
        iou = inter / (sa[:, None] + sb[None, :] - inter)
        valid = la_ref[0, 0, :][:, None] == lb_ref[0, 0, :][None, :]
        if tri or edge:
            rj = lax.broadcasted_iota(jnp.int32, inter.shape, 1)
        if tri:
            ri = lax.broadcasted_iota(jnp.int32, inter.shape, 0)
            valid &= ri < rj
        if edge:
            valid &= (b * T + rj) < n
        d = jnp.where(valid, iou, 0.0)
        csl = c_scr[:, pl.ds(b * T, T)]
        csl = jnp.maximum(csl, jnp.max(d, axis=0)[None, :])
        c_scr[:, pl.ds(b * T, T)] = csl
        ca = c_scr[0, pl.ds(a * T, T)]
        term = d * d - (ca * ca)[:, None]
        m_ref[0, 0, :] = jnp.maximum(m_ref[0, 0, :], jnp.max(term, axis=0))

    edge_col = nb * T > n

    @pl.when(a < b)
    def _():
        if edge_col:
            @pl.when(b < nb - 1)
            def _():
                tile(False, False)

            @pl.when(b == nb - 1)
            def _():
                tile(False, True)
        else:
            tile(False, False)

    @pl.when(a == b)
    def _():
        if edge_col:
            @pl.when(b < nb - 1)
            def _():
                tile(True, False)

            @pl.when(b == nb - 1)
            def _():
                tile(True, True)
        else:
            tile(True, False)

    @pl.when(a == nb - 1)
    def _():
        m_ref[...] = jnp.exp(-_SIGMA * m_ref[...])


def _to_bf16_body(x_ref, o_ref):
    o_ref[...] = x_ref[...].astype(jnp.bfloat16)


def _to_bf16(x, T):
    n, hw = x.shape
    nb = (n + T - 1) // T
    return pl.pallas_call(
        _to_bf16_body,
        grid=(nb,),
        in_specs=[pl.BlockSpec((T, hw), lambda i: (i, 0))],
        out_specs=pl.BlockSpec((T, hw), lambda i: (i, 0)),
        out_shape=jax.ShapeDtypeStruct((n, hw), jnp.bfloat16),
    )(x)


def _nms_core(masks, sums, labels, T):
    n, hw = masks.shape
    nb = (n + T - 1) // T
    P = nb * T
    grid = (nb, nb)
    sums3 = sums.reshape(nb, 1, T)
    labels3 = labels.reshape(nb, 1, T)
    mspec_a = pl.BlockSpec((T, hw), lambda b, a: (jnp.minimum(a, b), 0))
    mspec_b = pl.BlockSpec((T, hw), lambda b, a: (b, 0))
    vspec_a = pl.BlockSpec((1, 1, T), lambda b, a: (jnp.minimum(a, b), 0, 0))
    vspec_b = pl.BlockSpec((1, 1, T), lambda b, a: (b, 0, 0))
    params = pltpu.CompilerParams(dimension_semantics=("arbitrary", "arbitrary"))
    coeff = pl.pallas_call(
        functools.partial(_sweep_body, T, nb, n),
        grid=grid,
        in_specs=[mspec_a, mspec_b, vspec_a, vspec_b, vspec_a, vspec_b],
        out_specs=pl.BlockSpec((1, 1, T), lambda b, a: (b, 0, 0)),
        out_shape=jax.ShapeDtypeStruct((nb, 1, T), jnp.float32),
        scratch_shapes=[pltpu.VMEM((1, P), jnp.float32)],
        compiler_params=params,
    )(masks, masks, sums3, sums3, labels3, labels3)
    return coeff.reshape(P)


def kernel(seg_preds, cate_scores, cate_labels):
    n, h, w = seg_preds.shape
    hw = h * w
    seg_masks_b = seg_preds > _MASK_THR
    seg_masks_f = seg_masks_b.astype(jnp.float32)
    sum_masks = seg_masks_f.reshape(n, -1).sum(axis=1)
    seg_scores = (seg_preds * seg_masks_f).reshape(n, -1).sum(axis=1) / sum_masks
    cs = cate_scores * seg_scores
    sort_inds = jnp.argsort(-cs)

    T = 640
    P = ((n + T - 1) // T) * T
    pad = P - n

    seg_preds_s = jnp.take(seg_preds, sort_inds, axis=0)
    masks_b_s = jnp.take(seg_masks_b, sort_inds, axis=0)
    masks_s = masks_b_s.reshape(n, hw).astype(jnp.bfloat16)
    sums_p = jnp.pad(jnp.take(sum_masks, sort_inds), (0, pad),
                     constant_values=1.0)
    labels_p = jnp.pad(jnp.take(cate_labels, sort_inds), (0, pad),
                       constant_values=-1)
    coeff = _nms_core(masks_s, sums_p, labels_p, T)
    scores_s = jnp.take(cs, sort_inds)
    return (seg_preds_s,
            scores_s * coeff[:n],
            jnp.take(cate_labels, sort_inds))


# speedup vs baseline: 1.5116x; 1.0034x over previous
import functools

import jax
import jax.numpy as jnp
from jax import lax
from jax.experimental import pallas as pl
from jax.experimental.pallas import tpu as pltpu

_MASK_THR = 0.45
_SIGMA = 2.0


def _sweep_body(T, nb, n, ma_ref, mb_ref, sa_ref, sb_ref, la_ref, lb_ref,
                m_ref, c_scr):
    b = pl.program_id(0)
    a = pl.program_id(1)

    @pl.when(a == 0)
    def _():
        m_ref[...] = jnp.zeros_like(m_ref)
        c_scr[:, pl.ds(b * T, T)] = jnp.zeros((1, T), jnp.float32)

    def tile(tri, edge):
        inter = lax.dot_general(ma_ref[...], mb_ref[...],
                                (((1,), (1,)), ((), ())),
                                preferred_element_type=jnp.float32)
        sa = sa_ref[0, 0, :]
        sb = sb_ref[0, 0, :]
        iou = inter / (sa[:, None] + sb[None, :] - inter)
        valid = la_ref[0, 0, :][:, None] == lb_ref[0, 0, :][None, :]
        if tri or edge:
            rj = lax.broadcasted_iota(jnp.int32, inter.shape, 1)
        if tri:
            ri = lax.broadcasted_iota(jnp.int32, inter.shape, 0)
            valid &= ri < rj
        if edge:
            valid &= (b * T + rj) < n
        d = jnp.where(valid, iou, 0.0)
        csl = c_scr[:, pl.ds(b * T, T)]
        csl = jnp.maximum(csl, jnp.max(d, axis=0)[None, :])
        c_scr[:, pl.ds(b * T, T)] = csl
        ca = c_scr[0, pl.ds(a * T, T)]
        term = d * d - (ca * ca)[:, None]
        m_ref[0, 0, :] = jnp.maximum(m_ref[0, 0, :], jnp.max(term, axis=0))

    edge_col = nb * T > n

    @pl.when(a < b)
    def _():
        if edge_col:
            @pl.when(b < nb - 1)
            def _():
                tile(False, False)

            @pl.when(b == nb - 1)
            def _():
                tile(False, True)
        else:
            tile(False, False)

    @pl.when(a == b)
    def _():
        if edge_col:
            @pl.when(b < nb - 1)
            def _():
                tile(True, False)

            @pl.when(b == nb - 1)
            def _():
                tile(True, True)
        else:
            tile(True, False)

    @pl.when(a == nb - 1)
    def _():
        m_ref[...] = jnp.exp(-_SIGMA * m_ref[...])


def _to_bf16_body(x_ref, o_ref):
    o_ref[...] = x_ref[...].astype(jnp.bfloat16)


def _to_bf16(x, T):
    n, hw = x.shape
    nb = (n + T - 1) // T
    return pl.pallas_call(
        _to_bf16_body,
        grid=(nb,),
        in_specs=[pl.BlockSpec((T, hw), lambda i: (i, 0))],
        out_specs=pl.BlockSpec((T, hw), lambda i: (i, 0)),
        out_shape=jax.ShapeDtypeStruct((n, hw), jnp.bfloat16),
    )(x)


def _nms_core(masks, sums, labels, T):
    n, hw = masks.shape
    nb = (n + T - 1) // T
    P = nb * T
    grid = (nb, nb)
    sums3 = sums.reshape(nb, 1, T)
    labels3 = labels.reshape(nb, 1, T)
    mspec_a = pl.BlockSpec((T, hw), lambda b, a: (jnp.minimum(a, b), 0))
    mspec_b = pl.BlockSpec((T, hw), lambda b, a: (b, 0))
    vspec_a = pl.BlockSpec((1, 1, T), lambda b, a: (jnp.minimum(a, b), 0, 0))
    vspec_b = pl.BlockSpec((1, 1, T), lambda b, a: (b, 0, 0))
    params = pltpu.CompilerParams(dimension_semantics=("arbitrary", "arbitrary"))
    coeff = pl.pallas_call(
        functools.partial(_sweep_body, T, nb, n),
        grid=grid,
        in_specs=[mspec_a, mspec_b, vspec_a, vspec_b, vspec_a, vspec_b],
        out_specs=pl.BlockSpec((1, 1, T), lambda b, a: (b, 0, 0)),
        out_shape=jax.ShapeDtypeStruct((nb, 1, T), jnp.float32),
        scratch_shapes=[pltpu.VMEM((1, P), jnp.float32)],
        compiler_params=params,
    )(masks, masks, sums3, sums3, labels3, labels3)
    return coeff.reshape(P)


def kernel(seg_preds, cate_scores, cate_labels):
    n, h, w = seg_preds.shape
    hw = h * w
    seg_masks_b = seg_preds > _MASK_THR
    seg_masks_f = seg_masks_b.astype(jnp.float32)
    sum_masks = seg_masks_f.reshape(n, -1).sum(axis=1)
    seg_scores = (seg_preds * seg_masks_f).reshape(n, -1).sum(axis=1) / sum_masks
    cs = cate_scores * seg_scores
    neg_s, sort_inds, sums_s, labels_s = lax.sort(
        (-cs, lax.iota(jnp.int32, n), sum_masks, cate_labels),
        num_keys=1, is_stable=True)
    cs_s = -neg_s

    T = 640
    P = ((n + T - 1) // T) * T
    pad = P - n

    seg_preds_s = jnp.take(seg_preds, sort_inds, axis=0)
    masks_b_s = jnp.take(seg_masks_b, sort_inds, axis=0)
    masks_s = masks_b_s.reshape(n, hw).astype(jnp.bfloat16)
    sums_p = jnp.pad(sums_s, (0, pad), constant_values=1.0)
    labels_p = jnp.pad(labels_s, (0, pad), constant_values=-1)
    coeff = _nms_core(masks_s, sums_p, labels_p, T)
    return (seg_preds_s,
            cs_s * coeff[:n],
            labels_s)


# speedup vs baseline: 2.2362x; 1.4794x over previous
import functools

import jax
import jax.numpy as jnp
from jax import lax
from jax.experimental import pallas as pl
from jax.experimental.pallas import tpu as pltpu

_MASK_THR = 0.45
_SIGMA = 2.0


def _sweep_body(T, nb, n, ma_ref, mb_ref, sa_ref, sb_ref, la_ref, lb_ref,
                m_ref, c_scr):
    b = pl.program_id(0)
    a = pl.program_id(1)

    @pl.when(a == 0)
    def _():
        m_ref[...] = jnp.zeros_like(m_ref)
        c_scr[:, pl.ds(b * T, T)] = jnp.zeros((1, T), jnp.float32)

    def tile(tri, edge):
        inter = lax.dot_general(ma_ref[...], mb_ref[...],
                                (((1,), (1,)), ((), ())),
                                preferred_element_type=jnp.float32)
        sa = sa_ref[0, 0, :]
        sb = sb_ref[0, 0, :]
        iou = inter / (sa[:, None] + sb[None, :] - inter)
        valid = la_ref[0, 0, :][:, None] == lb_ref[0, 0, :][None, :]
        if tri or edge:
            rj = lax.broadcasted_iota(jnp.int32, inter.shape, 1)
        if tri:
            ri = lax.broadcasted_iota(jnp.int32, inter.shape, 0)
            valid &= ri < rj
        if edge:
            valid &= (b * T + rj) < n
        d = jnp.where(valid, iou, 0.0)
        csl = c_scr[:, pl.ds(b * T, T)]
        csl = jnp.maximum(csl, jnp.max(d, axis=0)[None, :])
        c_scr[:, pl.ds(b * T, T)] = csl
        ca = c_scr[0, pl.ds(a * T, T)]
        term = d * d - (ca * ca)[:, None]
        m_ref[0, 0, :] = jnp.maximum(m_ref[0, 0, :], jnp.max(term, axis=0))

    edge_col = nb * T > n

    @pl.when(a < b)
    def _():
        if edge_col:
            @pl.when(b < nb - 1)
            def _():
                tile(False, False)

            @pl.when(b == nb - 1)
            def _():
                tile(False, True)
        else:
            tile(False, False)

    @pl.when(a == b)
    def _():
        if edge_col:
            @pl.when(b < nb - 1)
            def _():
                tile(True, False)

            @pl.when(b == nb - 1)
            def _():
                tile(True, True)
        else:
            tile(True, False)

    @pl.when(a == nb - 1)
    def _():
        m_ref[...] = jnp.exp(-_SIGMA * m_ref[...])


def _to_bf16_body(x_ref, o_ref):
    o_ref[...] = x_ref[...].astype(jnp.bfloat16)


def _to_bf16(x, T):
    n, hw = x.shape
    nb = (n + T - 1) // T
    return pl.pallas_call(
        _to_bf16_body,
        grid=(nb,),
        in_specs=[pl.BlockSpec((T, hw), lambda i: (i, 0))],
        out_specs=pl.BlockSpec((T, hw), lambda i: (i, 0)),
        out_shape=jax.ShapeDtypeStruct((n, hw), jnp.bfloat16),
    )(x)


def _nms_core(masks, sums, labels, T):
    n, hw = masks.shape
    nb = (n + T - 1) // T
    P = nb * T
    grid = (nb, nb)
    sums3 = sums.reshape(nb, 1, T)
    labels3 = labels.reshape(nb, 1, T)
    mspec_a = pl.BlockSpec((T, hw), lambda b, a: (jnp.minimum(a, b), 0))
    mspec_b = pl.BlockSpec((T, hw), lambda b, a: (b, 0))
    vspec_a = pl.BlockSpec((1, 1, T), lambda b, a: (jnp.minimum(a, b), 0, 0))
    vspec_b = pl.BlockSpec((1, 1, T), lambda b, a: (b, 0, 0))
    params = pltpu.CompilerParams(dimension_semantics=("arbitrary", "arbitrary"))
    coeff = pl.pallas_call(
        functools.partial(_sweep_body, T, nb, n),
        grid=grid,
        in_specs=[mspec_a, mspec_b, vspec_a, vspec_b, vspec_a, vspec_b],
        out_specs=pl.BlockSpec((1, 1, T), lambda b, a: (b, 0, 0)),
        out_shape=jax.ShapeDtypeStruct((nb, 1, T), jnp.float32),
        scratch_shapes=[pltpu.VMEM((1, P), jnp.float32)],
        compiler_params=params,
    )(masks, masks, sums3, sums3, labels3, labels3)
    return coeff.reshape(P)


def kernel(seg_preds, cate_scores, cate_labels):
    n, h, w = seg_preds.shape
    hw = h * w
    seg_masks_b = seg_preds > _MASK_THR
    seg_masks_f = seg_masks_b.astype(jnp.float32)
    sum_masks = seg_masks_f.reshape(n, -1).sum(axis=1)
    seg_scores = (seg_preds * seg_masks_f).reshape(n, -1).sum(axis=1) / sum_masks
    cs = cate_scores * seg_scores
    neg_s, sort_inds, sums_s, labels_s = lax.sort(
        (-cs, lax.iota(jnp.int32, n), sum_masks, cate_labels),
        num_keys=1, is_stable=True)
    cs_s = -neg_s

    T = 640
    P = ((n + T - 1) // T) * T
    pad = P - n

    seg_preds_s = jnp.take(seg_preds.reshape(n, hw), sort_inds,
                           axis=0).reshape(n, h, w)
    masks_b_s = jnp.take(seg_masks_b, sort_inds, axis=0)
    masks_s = masks_b_s.reshape(n, hw).astype(jnp.bfloat16)
    sums_p = jnp.pad(sums_s, (0, pad), constant_values=1.0)
    labels_p = jnp.pad(labels_s, (0, pad), constant_values=-1)
    coeff = _nms_core(masks_s, sums_p, labels_p, T)
    return (seg_preds_s,
            cs_s * coeff[:n],
            labels_s)


# speedup vs baseline: 2.2480x; 1.0053x over previous
import functools

import jax
import jax.numpy as jnp
from jax import lax
from jax.experimental import pallas as pl
from jax.experimental.pallas import tpu as pltpu
from jax.experimental.pallas import tpu_sc as plsc

_MASK_THR = 0.45
_SIGMA = 2.0


def _sweep_body(T, nb, n, ma_ref, mb_ref, sa_ref, sb_ref, la_ref, lb_ref,
                m_ref, c_scr):
    b = pl.program_id(0)
    a = pl.program_id(1)

    @pl.when(a == 0)
    def _():
        m_ref[...] = jnp.zeros_like(m_ref)
        c_scr[:, pl.ds(b * T, T)] = jnp.zeros((1, T), jnp.float32)

    def tile(tri, edge):
        inter = lax.dot_general(ma_ref[...], mb_ref[...],
                                (((1,), (1,)), ((), ())),
                                preferred_element_type=jnp.float32)
        sa = sa_ref[0, 0, :]
        sb = sb_ref[0, 0, :]
        iou = inter / (sa[:, None] + sb[None, :] - inter)
        valid = la_ref[0, 0, :][:, None] == lb_ref[0, 0, :][None, :]
        if tri or edge:
            rj = lax.broadcasted_iota(jnp.int32, inter.shape, 1)
        if tri:
            ri = lax.broadcasted_iota(jnp.int32, inter.shape, 0)
            valid &= ri < rj
        if edge:
            valid &= (b * T + rj) < n
        d = jnp.where(valid, iou, 0.0)
        csl = c_scr[:, pl.ds(b * T, T)]
        csl = jnp.maximum(csl, jnp.max(d, axis=0)[None, :])
        c_scr[:, pl.ds(b * T, T)] = csl
        ca = c_scr[0, pl.ds(a * T, T)]
        term = d * d - (ca * ca)[:, None]
        m_ref[0, 0, :] = jnp.maximum(m_ref[0, 0, :], jnp.max(term, axis=0))

    edge_col = nb * T > n

    @pl.when(a < b)
    def _():
        if edge_col:
            @pl.when(b < nb - 1)
            def _():
                tile(False, False)

            @pl.when(b == nb - 1)
            def _():
                tile(False, True)
        else:
            tile(False, False)

    @pl.when(a == b)
    def _():
        if edge_col:
            @pl.when(b < nb - 1)
            def _():
                tile(True, False)

            @pl.when(b == nb - 1)
            def _():
                tile(True, True)
        else:
            tile(True, False)

    @pl.when(a == nb - 1)
    def _():
        m_ref[...] = jnp.exp(-_SIGMA * m_ref[...])


def _to_bf16_body(x_ref, o_ref):
    o_ref[...] = x_ref[...].astype(jnp.bfloat16)


def _to_bf16(x, T):
    n, hw = x.shape
    nb = (n + T - 1) // T
    return pl.pallas_call(
        _to_bf16_body,
        grid=(nb,),
        in_specs=[pl.BlockSpec((T, hw), lambda i: (i, 0))],
        out_specs=pl.BlockSpec((T, hw), lambda i: (i, 0)),
        out_shape=jax.ShapeDtypeStruct((n, hw), jnp.bfloat16),
    )(x)


def _sc_gather_rows(table, idx):
    n, hw = table.shape
    info = plsc.get_sparse_core_info()
    nw = info.num_cores * info.num_subcores
    ch = 40
    b_per_w = ((n + nw - 1) // nw + ch - 1) // ch * ch
    mesh = plsc.VectorSubcoreMesh(core_axis_name="c", subcore_axis_name="s")

    @functools.partial(
        pl.kernel, mesh=mesh,
        out_type=jax.ShapeDtypeStruct((n, hw), jnp.float32),
        scratch_types=[
            pltpu.VMEM((ch,), jnp.int32),
            pltpu.VMEM((ch, hw), jnp.float32),
            pltpu.SemaphoreType.DMA,
        ],
    )
    def k(table_hbm, idx_hbm, out_hbm, idx_v, rows_v, sem):
        wid = lax.axis_index("s") * info.num_cores + lax.axis_index("c")
        base = wid * b_per_w
        for t in range(b_per_w // ch):
            start = base + t * ch

            @pl.when(start < n)
            def _():
                pltpu.sync_copy(idx_hbm.at[pl.ds(start, ch)], idx_v)
                pltpu.async_copy(table_hbm.at[idx_v], rows_v, sem).wait()
                pltpu.sync_copy(rows_v, out_hbm.at[pl.ds(start, ch)])

    return k(table, idx)


def _nms_core(masks, sums, labels, T):
    n, hw = masks.shape
    nb = (n + T - 1) // T
    P = nb * T
    grid = (nb, nb)
    sums3 = sums.reshape(nb, 1, T)
    labels3 = labels.reshape(nb, 1, T)
    mspec_a = pl.BlockSpec((T, hw), lambda b, a: (jnp.minimum(a, b), 0))
    mspec_b = pl.BlockSpec((T, hw), lambda b, a: (b, 0))
    vspec_a = pl.BlockSpec((1, 1, T), lambda b, a: (jnp.minimum(a, b), 0, 0))
    vspec_b = pl.BlockSpec((1, 1, T), lambda b, a: (b, 0, 0))
    params = pltpu.CompilerParams(dimension_semantics=("arbitrary", "arbitrary"))
    coeff = pl.pallas_call(
        functools.partial(_sweep_body, T, nb, n),
        grid=grid,
        in_specs=[mspec_a, mspec_b, vspec_a, vspec_b, vspec_a, vspec_b],
        out_specs=pl.BlockSpec((1, 1, T), lambda b, a: (b, 0, 0)),
        out_shape=jax.ShapeDtypeStruct((nb, 1, T), jnp.float32),
        scratch_shapes=[pltpu.VMEM((1, P), jnp.float32)],
        compiler_params=params,
    )(masks, masks, sums3, sums3, labels3, labels3)
    return coeff.reshape(P)


def kernel(seg_preds, cate_scores, cate_labels):
    n, h, w = seg_preds.shape
    hw = h * w
    seg_masks_b = seg_preds > _MASK_THR
    seg_masks_f = seg_masks_b.astype(jnp.float32)
    sum_masks = seg_masks_f.reshape(n, -1).sum(axis=1)
    seg_scores = (seg_preds * seg_masks_f).reshape(n, -1).sum(axis=1) / sum_masks
    cs = cate_scores * seg_scores
    neg_s, sort_inds, sums_s, labels_s = lax.sort(
        (-cs, lax.iota(jnp.int32, n), sum_masks, cate_labels),
        num_keys=1, is_stable=True)
    cs_s = -neg_s

    T = 640
    P = ((n + T - 1) // T) * T
    pad = P - n

    seg_preds_s = _sc_gather_rows(seg_preds.reshape(n, hw),
                                  sort_inds).reshape(n, h, w)
    masks_b_s = jnp.take(seg_masks_b, sort_inds, axis=0)
    masks_s = masks_b_s.reshape(n, hw).astype(jnp.bfloat16)
    sums_p = jnp.pad(sums_s, (0, pad), constant_values=1.0)
    labels_p = jnp.pad(labels_s, (0, pad), constant_values=-1)
    coeff = _nms_core(masks_s, sums_p, labels_p, T)
    return (seg_preds_s,
            cs_s * coeff[:n],
            labels_s)
